# Initial kernel scaffold; baseline (speedup 1.0000x reference)
import jax
import jax.numpy as jnp
from jax.experimental import pallas as pl


def kernel(users, items, edge_index, user_emb, item_emb, Wmu, bmu, Wlv, blv, Wdec, bdec, attW, attb, eps):
    raise NotImplementedError("write your pallas kernel here")


# speedup vs baseline: 4.8243x; 4.8243x over previous
import functools

import jax
import jax.numpy as jnp
from jax import lax
from jax.experimental import pallas as pl
from jax.experimental.pallas import tpu as pltpu
from jax.experimental.pallas import tpu_sc as plsc

NC = 2
NS = 16
C = 80


def _sc_mesh():
    return plsc.VectorSubcoreMesh(
        core_axis_name="c", subcore_axis_name="s", num_cores=NC, num_subcores=NS)


_SC_PARAMS = pltpu.CompilerParams(use_tc_tiling_on_sc=False,
                                  needs_layout_passes=False)


def _degree_call(row_arr, N, E):
    HALF = N // 2
    HP = ((HALF + NS - 1) // NS + 7) // 8 * 8
    PH = HP * NS
    DUMP = HALF
    EPT = E // NS
    NCH = EPT // C
    assert E % NS == 0 and EPT % C == 0 and HALF < PH
    ZR = 98
    assert HP % ZR == 0

    @functools.partial(
        pl.kernel,
        out_type=jax.ShapeDtypeStruct((2 * PH, 16), jnp.float32),
        mesh=_sc_mesh(),
        compiler_params=_SC_PARAMS,
        scratch_types=[
            pltpu.VMEM((C,), jnp.int32),
            pltpu.VMEM((C, 16), jnp.float32),
            pltpu.VMEM((ZR, 16), jnp.float32),
            pltpu.VMEM_SHARED((PH, 16), jnp.float32),
        ],
    )
    def deg_kernel(row_hbm, out_hbm, locv, onesv, zbuf, table):
        c = lax.axis_index("c")
        s = lax.axis_index("s")
        base_row = c * HALF

        def fill(i, _):
            zbuf[i, :] = jnp.zeros((16,), jnp.float32)
            onesv[jnp.minimum(i, C - 1), :] = jnp.ones((16,), jnp.float32)
            return 0
        lax.fori_loop(0, ZR, fill, 0)
        for k in range(HP // ZR):
            pltpu.sync_copy(zbuf, table.at[pl.ds(s * HP + k * ZR, ZR)])
        plsc.subcore_barrier()

        def chunk(g, _):
            pltpu.sync_copy(row_hbm.at[pl.ds(s * EPT + g * C, C)], locv)
            for j in range(C // 16):
                v = locv[pl.ds(j * 16, 16)]
                lv = v - base_row
                ok = (lv >= 0) & (lv < HALF)
                locv[pl.ds(j * 16, 16)] = jnp.where(ok, lv, DUMP)
            pltpu.sync_copy(onesv, table.at[locv], add=True)
            return 0
        lax.fori_loop(0, NCH, chunk, 0)
        plsc.subcore_barrier()
        pltpu.sync_copy(table.at[pl.ds(s * HP, HP)],
                        out_hbm.at[pl.ds(c * PH + s * HP, HP)])

    return deg_kernel(row_arr), PH


def _spmm_call(g_tab, row_arr, col_arr, N, E, D):
    HALF = N // 2
    HP = ((HALF + NS - 1) // NS + 7) // 8 * 8
    PH = HP * NS
    DUMP = HALF
    EPT = E // NS
    NCH = EPT // C
    ZR = 98
    assert HP % ZR == 0

    @functools.partial(
        pl.kernel,
        out_type=jax.ShapeDtypeStruct((2 * PH, D), jnp.float32),
        mesh=_sc_mesh(),
        compiler_params=_SC_PARAMS,
        scratch_types=[
            pltpu.VMEM((C,), jnp.int32),
            pltpu.VMEM((C,), jnp.int32),
            pltpu.VMEM((C, D), jnp.float32),
            pltpu.VMEM((ZR, D), jnp.float32),
            pltpu.VMEM_SHARED((PH, D), jnp.float32),
            pltpu.SemaphoreType.DMA,
        ],
    )
    def spmm_kernel(g_hbm, row_hbm, col_hbm, out_hbm, colv, locv, gbuf, zbuf,
                    table, sem):
        c = lax.axis_index("c")
        s = lax.axis_index("s")
        base_row = c * HALF

        def fill(i, _):
            for q in range(D // 16):
                zbuf[i, pl.ds(q * 16, 16)] = jnp.zeros((16,), jnp.float32)
            return 0
        lax.fori_loop(0, ZR, fill, 0)
        for k in range(HP // ZR):
            pltpu.sync_copy(zbuf, table.at[pl.ds(s * HP + k * ZR, ZR)])
        plsc.subcore_barrier()

        def chunk(g, _):
            e0 = s * EPT + g * C
            pltpu.sync_copy(col_hbm.at[pl.ds(e0, C)], colv)
            pltpu.async_copy(g_hbm.at[colv], gbuf, sem).wait()
            pltpu.sync_copy(row_hbm.at[pl.ds(e0, C)], locv)
            for j in range(C // 16):
                v = locv[pl.ds(j * 16, 16)]
                lv = v - base_row
                ok = (lv >= 0) & (lv < HALF)
                locv[pl.ds(j * 16, 16)] = jnp.where(ok, lv, DUMP)
            pltpu.sync_copy(gbuf, table.at[locv], add=True)
            return 0
        lax.fori_loop(0, NCH, chunk, 0)
        plsc.subcore_barrier()
        pltpu.sync_copy(table.at[pl.ds(s * HP, HP)],
                        out_hbm.at[pl.ds(c * PH + s * HP, HP)])

    return spmm_kernel(g_tab, row_arr, col_arr)


def _scores_call(fin, users, items, user_num, N, D, B):
    NW = NC * NS
    P = B // NW
    assert B % NW == 0 and P % 16 == 0

    @functools.partial(
        pl.kernel,
        out_type=jax.ShapeDtypeStruct((B,), jnp.float32),
        mesh=_sc_mesh(),
        compiler_params=_SC_PARAMS,
        scratch_types=[
            pltpu.VMEM((P,), jnp.int32),
            pltpu.VMEM((P,), jnp.int32),
            pltpu.VMEM((P, D), jnp.float32),
            pltpu.VMEM((P, D), jnp.float32),
            pltpu.VMEM((P,), jnp.float32),
            pltpu.SemaphoreType.DMA,
        ],
    )
    def score_kernel(fin_hbm, u_hbm, i_hbm, out_hbm, uidx, iidx, ubuf, vbuf,
                     svec, sem):
        c = lax.axis_index("c")
        s = lax.axis_index("s")
        wid = s * NC + c
        base = wid * P
        pltpu.sync_copy(u_hbm.at[pl.ds(base, P)], uidx)
        pltpu.sync_copy(i_hbm.at[pl.ds(base, P)], iidx)
        for j in range(P // 16):
            iidx[pl.ds(j * 16, 16)] = iidx[pl.ds(j * 16, 16)] + user_num
        pltpu.async_copy(fin_hbm.at[uidx], ubuf, sem).wait()
        pltpu.async_copy(fin_hbm.at[iidx], vbuf, sem).wait()
        iota = lax.broadcasted_iota(jnp.int32, (16,), 0)

        def group(j, _):
            acc = jnp.zeros((16,), jnp.float32)
            for k in range(16):
                p = j * 16 + k
                t = ubuf[p, pl.ds(0, 16)] * vbuf[p, pl.ds(0, 16)]
                for q in range(1, D // 16):
                    t = t + ubuf[p, pl.ds(q * 16, 16)] * vbuf[p, pl.ds(q * 16, 16)]
                acc = jnp.where(iota == k, jnp.sum(t), acc)
            svec[pl.ds(j * 16, 16)] = acc
            return 0
        lax.fori_loop(0, P // 16, group, 0)
        pltpu.sync_copy(svec, out_hbm.at[pl.ds(base, P)])

    return score_kernel(fin, users, items)


def _g0_call(deg16, x0, N, D):
    R = 2000
    GRID = N // R

    def body(deg_ref, x_ref, g_ref):
        d = lax.rsqrt(deg_ref[:, 0:1] + 1e-10)
        g_ref[...] = x_ref[...] * d

    return pl.pallas_call(
        body,
        grid=(GRID,),
        in_specs=[
            pl.BlockSpec((R, 16), lambda i: (i, 0)),
            pl.BlockSpec((R, D), lambda i: (i, 0)),
        ],
        out_specs=pl.BlockSpec((R, D), lambda i: (i, 0)),
        out_shape=jax.ShapeDtypeStruct((N, D), jnp.float32),
    )(deg16, x0)


def _layer_call(deg16, acc, epsl, Wmu, bmu, Wlv, blv, Wdec, bdec, N, D,
                need_g):
    R = 2000
    GRID = N // R

    def body(deg_ref, acc_ref, eps_ref, wmu_ref, bmu_ref, wlv_ref, blv_ref,
             wdec_ref, bdec_ref, x_ref, g_ref, part_ref):
        d = lax.rsqrt(deg_ref[:, 0:1] + 1e-10)
        x = acc_ref[...] * d
        x_ref[...] = x
        g_ref[...] = x * d
        mu = jnp.dot(x, wmu_ref[...], preferred_element_type=jnp.float32) \
            + bmu_ref[...]
        lgv = jnp.dot(x, wlv_ref[...], preferred_element_type=jnp.float32) \
            + blv_ref[...]
        elv = jnp.exp(lgv)
        std = jnp.exp(0.5 * lgv)
        z = mu + eps_ref[...] * std
        recon = jnp.dot(z, wdec_ref[...], preferred_element_type=jnp.float32) \
            + bdec_ref[...]
        i = pl.program_id(0)
        part_ref[i, 0] = jnp.sum((recon - x) ** 2)
        part_ref[i, 1] = jnp.sum(1.0 + lgv - mu * mu - elv)

    w_spec = pl.BlockSpec((D, D), lambda i: (0, 0))
    b_spec = pl.BlockSpec((1, D), lambda i: (0, 0))
    x_arr, g_arr, parts = pl.pallas_call(
        body,
        grid=(GRID,),
        in_specs=[
            pl.BlockSpec((R, 16), lambda i: (i, 0)),
            pl.BlockSpec((R, D), lambda i: (i, 0)),
            pl.BlockSpec((R, D), lambda i: (i, 0)),
            w_spec, b_spec, w_spec, b_spec, w_spec, b_spec,
        ],
        out_specs=[
            pl.BlockSpec((R, D), lambda i: (i, 0)),
            pl.BlockSpec((R, D), lambda i: (i, 0)),
            pl.BlockSpec((GRID, 2), lambda i: (0, 0), memory_space=pltpu.SMEM),
        ],
        out_shape=[
            jax.ShapeDtypeStruct((N, D), jnp.float32),
            jax.ShapeDtypeStruct((N, D), jnp.float32),
            jax.ShapeDtypeStruct((GRID, 2), jnp.float32),
        ],
    )(deg16, acc, epsl, Wmu, bmu.reshape(1, D), Wlv, blv.reshape(1, D),
      Wdec, bdec.reshape(1, D))
    del need_g
    return x_arr, g_arr, parts


def _final_call(deg16, acc3, x0, x1, x2, eps2, eps3, W2, W3, attW, attb, N, D):
    R = 2000
    GRID = N // R
    LP1 = 4

    def vae(x, eps_blk, wmu, bmu, wlv, blv, wdec, bdec):
        mu = jnp.dot(x, wmu, preferred_element_type=jnp.float32) + bmu
        lgv = jnp.dot(x, wlv, preferred_element_type=jnp.float32) + blv
        elv = jnp.exp(lgv)
        z = mu + eps_blk * jnp.exp(0.5 * lgv)
        recon = jnp.dot(z, wdec, preferred_element_type=jnp.float32) + bdec
        return jnp.sum((recon - x) ** 2), jnp.sum(1.0 + lgv - mu * mu - elv)

    def body(deg_ref, acc_ref, x0_ref, x1_ref, x2_ref, eps2_ref, eps3_ref,
             wmu2, bmu2, wlv2, blv2, wdec2, bdec2,
             wmu3, bmu3, wlv3, blv3, wdec3, bdec3,
             attw_ref, attb_ref, fin_ref, part_ref):
        d = lax.rsqrt(deg_ref[:, 0:1] + 1e-10)
        x3 = acc_ref[...] * d
        r2, t2 = vae(x3, eps2_ref[...], wmu2[...], bmu2[...], wlv2[...],
                     blv2[...], wdec2[...], bdec2[...])
        r3, t3 = vae(x3, eps3_ref[...], wmu3[...], bmu3[...], wlv3[...],
                     blv3[...], wdec3[...], bdec3[...])
        i = pl.program_id(0)
        part_ref[i, 0] = r2
        part_ref[i, 1] = t2
        part_ref[i, 2] = r3
        part_ref[i, 3] = t3
        e0, e1, e2 = x0_ref[...], x1_ref[...], x2_ref[...]
        flat = jnp.concatenate([e0, e1, e2, x3], axis=1)
        logits = jnp.dot(flat, attw_ref[...],
                         preferred_element_type=jnp.float32) + attb_ref[...]
        m = jnp.max(logits, axis=1, keepdims=True)
        p = jnp.exp(logits - m)
        p = p / jnp.sum(p, axis=1, keepdims=True)
        fin_ref[...] = (e0 * p[:, 0:1] + e1 * p[:, 1:2]
                        + e2 * p[:, 2:3] + x3 * p[:, 3:4])

    w_spec = pl.BlockSpec((D, D), lambda i: (0, 0))
    b_spec = pl.BlockSpec((1, D), lambda i: (0, 0))
    r_spec = pl.BlockSpec((R, D), lambda i: (i, 0))
    (Wmu2, bmu2, Wlv2, blv2, Wdec2, bdec2) = W2
    (Wmu3, bmu3, Wlv3, blv3, Wdec3, bdec3) = W3
    fin, parts = pl.pallas_call(
        body,
        grid=(GRID,),
        in_specs=[
            pl.BlockSpec((R, 16), lambda i: (i, 0)),
            r_spec, r_spec, r_spec, r_spec, r_spec, r_spec,
            w_spec, b_spec, w_spec, b_spec, w_spec, b_spec,
            w_spec, b_spec, w_spec, b_spec, w_spec, b_spec,
            pl.BlockSpec((LP1 * D, LP1), lambda i: (0, 0)),
            pl.BlockSpec((1, LP1), lambda i: (0, 0)),
        ],
        out_specs=[
            r_spec,
            pl.BlockSpec((GRID, 4), lambda i: (0, 0), memory_space=pltpu.SMEM),
        ],
        out_shape=[
            jax.ShapeDtypeStruct((N, D), jnp.float32),
            jax.ShapeDtypeStruct((GRID, 4), jnp.float32),
        ],
    )(deg16, acc3, x0, x1, x2, eps2, eps3,
      Wmu2, bmu2.reshape(1, D), Wlv2, blv2.reshape(1, D), Wdec2,
      bdec2.reshape(1, D),
      Wmu3, bmu3.reshape(1, D), Wlv3, blv3.reshape(1, D), Wdec3,
      bdec3.reshape(1, D),
      attW, attb.reshape(1, LP1))
    return fin, parts


def kernel(users, items, edge_index, user_emb, item_emb, Wmu, bmu, Wlv, blv,
           Wdec, bdec, attW, attb, eps):
    user_num = user_emb.shape[0]
    N = user_num + item_emb.shape[0]
    D = user_emb.shape[1]
    E = edge_index.shape[1]
    B = users.shape[0]
    L = Wmu.shape[0] - 1
    HALF = N // 2
    HP = ((HALF + NS - 1) // NS + 7) // 8 * 8
    PH = HP * NS

    row = edge_index[0]
    col = edge_index[1]
    x0 = jnp.concatenate([user_emb, item_emb], axis=0)

    degpad, _ = _degree_call(row, N, E)
    deg16 = jnp.concatenate([degpad[0:HALF], degpad[PH:PH + HALF]], axis=0)

    g = _g0_call(deg16, x0, N, D)
    xs = [x0]
    parts = []
    for l in range(1, L):
        accpad = _spmm_call(g, row, col, N, E, D)
        acc = jnp.concatenate([accpad[0:HALF], accpad[PH:PH + HALF]], axis=0)
        x_l, g, p = _layer_call(deg16, acc, eps[l - 1], Wmu[l - 1], bmu[l - 1],
                                Wlv[l - 1], blv[l - 1], Wdec[l - 1],
                                bdec[l - 1], N, D, True)
        xs.append(x_l)
        parts.append(p)
    accpad = _spmm_call(g, row, col, N, E, D)
    acc3 = jnp.concatenate([accpad[0:HALF], accpad[PH:PH + HALF]], axis=0)
    W2 = (Wmu[L - 1], bmu[L - 1], Wlv[L - 1], blv[L - 1], Wdec[L - 1],
          bdec[L - 1])
    W3 = (Wmu[L], bmu[L], Wlv[L], blv[L], Wdec[L], bdec[L])
    fin, fparts = _final_call(deg16, acc3, xs[0], xs[1], xs[2], eps[L - 1],
                              eps[L], W2, W3, attW, attb, N, D)

    scores = _scores_call(fin, users, items, user_num, N, D, B)

    ND = float(N * D)
    losses = []
    for p in parts:
        rsum = jnp.sum(p[:, 0])
        tsum = jnp.sum(p[:, 1])
        losses.append(rsum / ND - 0.5 * tsum / ND)
    fr = jnp.sum(fparts, axis=0)
    losses.append(fr[0] / ND - 0.5 * fr[1] / ND)
    losses.append(fr[2] / ND - 0.5 * fr[3] / ND)
    loss = jnp.mean(jnp.stack(losses))
    return scores, loss


# speedup vs baseline: 8.4982x; 1.7615x over previous
import functools

import jax
import jax.numpy as jnp
from jax import lax
from jax.experimental import pallas as pl
from jax.experimental.pallas import tpu as pltpu
from jax.experimental.pallas import tpu_sc as plsc

NC = 2
NS = 16
C = 80


def _sc_mesh():
    return plsc.VectorSubcoreMesh(
        core_axis_name="c", subcore_axis_name="s", num_cores=NC, num_subcores=NS)


_SC_PARAMS = pltpu.CompilerParams(use_tc_tiling_on_sc=False,
                                  needs_layout_passes=False)


def _degree_call(row_arr, N, E):
    HALF = N // 2
    HP = ((HALF + NS - 1) // NS + 7) // 8 * 8
    PH = HP * NS
    DUMP = HALF
    EPT = E // NS
    NCH = EPT // C
    assert E % NS == 0 and EPT % C == 0 and HALF < PH
    NBUF = 5
    assert NCH % NBUF == 0

    @functools.partial(
        pl.kernel,
        out_type=jax.ShapeDtypeStruct((2 * PH, 16), jnp.float32),
        mesh=_sc_mesh(),
        compiler_params=_SC_PARAMS,
        scratch_types=[
            pltpu.VMEM((EPT,), jnp.int32),
            pltpu.VMEM((C, 16), jnp.float32),
            pltpu.VMEM((C, 16), jnp.float32),
            pltpu.VMEM((NBUF, C), jnp.int32),
            pltpu.VMEM_SHARED((PH, 16), jnp.float32),
            pltpu.SemaphoreType.DMA((NBUF,)),
        ],
    )
    def deg_kernel(row_hbm, out_hbm, rowblk, onesv, zbuf, locv, table, ssem):
        c = lax.axis_index("c")
        s = lax.axis_index("s")
        base_row = c * HALF

        def fill(i, _):
            zbuf[i, :] = jnp.zeros((16,), jnp.float32)
            onesv[i, :] = jnp.ones((16,), jnp.float32)
            return 0
        lax.fori_loop(0, C, fill, 0)
        nfull, rem = HP // C, HP % C
        for t in range(nfull):
            pltpu.sync_copy(zbuf, table.at[pl.ds(s * HP + t * C, C)])
        if rem:
            pltpu.sync_copy(zbuf.at[pl.ds(0, rem)],
                            table.at[pl.ds(s * HP + nfull * C, rem)])
        plsc.subcore_barrier()

        pltpu.sync_copy(row_hbm.at[pl.ds(s * EPT, EPT)], rowblk)

        def fixup(slot, k):
            for j in range(C // 16):
                v = rowblk[pl.ds(k * C + j * 16, 16)]
                lv = v - base_row
                ok = (lv >= 0) & (lv < HALF)
                locv[slot, pl.ds(j * 16, 16)] = jnp.where(ok, lv, DUMP)

        def round_body(i, _):
            for b in range(NBUF):
                k = i * NBUF + b

                @pl.when(i > 0)
                def _():
                    pltpu.make_async_copy(onesv, table.at[locv.at[b]],
                                          ssem.at[b]).wait()
                fixup(b, k)
                pltpu.async_copy(onesv, table.at[locv.at[b]], ssem.at[b],
                                 add=True)
            return 0
        lax.fori_loop(0, NCH // NBUF, round_body, 0)
        for b in range(NBUF):
            pltpu.make_async_copy(onesv, table.at[locv.at[b]],
                                  ssem.at[b]).wait()
        plsc.subcore_barrier()
        pltpu.sync_copy(table.at[pl.ds(s * HP, HP)],
                        out_hbm.at[pl.ds(c * PH + s * HP, HP)])

    return deg_kernel(row_arr), PH


def _spmm_call(g_tab, row_arr, col_arr, N, E, D):
    HALF = N // 2
    HP = ((HALF + NS - 1) // NS + 7) // 8 * 8
    PH = HP * NS
    DUMP = HALF
    EPT = E // NS
    NCH = EPT // C
    NBUF = 5
    assert NCH % NBUF == 0

    @functools.partial(
        pl.kernel,
        out_type=jax.ShapeDtypeStruct((2 * PH, D), jnp.float32),
        mesh=_sc_mesh(),
        compiler_params=_SC_PARAMS,
        scratch_types=[
            pltpu.VMEM((NBUF, C, D), jnp.float32),
            pltpu.VMEM((NBUF, C), jnp.int32),
            pltpu.VMEM((NBUF, C), jnp.int32),
            pltpu.VMEM_SHARED((PH, D), jnp.float32),
            pltpu.SemaphoreType.DMA((NBUF,)),
            pltpu.SemaphoreType.DMA((NBUF,)),
        ],
    )
    def spmm_kernel(g_hbm, row_hbm, col_hbm, out_hbm, gbuf, colv, locv,
                    table, gsem, isem):
        c = lax.axis_index("c")
        s = lax.axis_index("s")
        base_row = c * HALF

        def fillz(i, _):
            for q in range(D // 16):
                gbuf[0, i, pl.ds(q * 16, 16)] = jnp.zeros((16,), jnp.float32)
            return 0
        lax.fori_loop(0, C, fillz, 0)
        nfull, rem = HP // C, HP % C
        for t in range(nfull):
            pltpu.sync_copy(gbuf.at[0], table.at[pl.ds(s * HP + t * C, C)])
        if rem:
            pltpu.sync_copy(gbuf.at[0].at[pl.ds(0, rem)],
                            table.at[pl.ds(s * HP + nfull * C, rem)])
        plsc.subcore_barrier()

        def issue_idx(slot, k):
            e0 = s * EPT + k * C
            pltpu.async_copy(row_hbm.at[pl.ds(e0, C)], locv.at[slot],
                             isem.at[slot])
            pltpu.async_copy(col_hbm.at[pl.ds(e0, C)], colv.at[slot],
                             isem.at[slot])

        def wait_idx(slot, k):
            e0 = s * EPT + k * C
            pltpu.make_async_copy(row_hbm.at[pl.ds(e0, C)], locv.at[slot],
                                  isem.at[slot]).wait()
            pltpu.make_async_copy(col_hbm.at[pl.ds(e0, C)], colv.at[slot],
                                  isem.at[slot]).wait()

        def fixup(slot):
            for j in range(C // 16):
                v = locv[slot, pl.ds(j * 16, 16)]
                lv = v - base_row
                ok = (lv >= 0) & (lv < HALF)
                locv[slot, pl.ds(j * 16, 16)] = jnp.where(ok, lv, DUMP)

        for b in range(NBUF):
            issue_idx(b, b)
        for b in range(NBUF - 1):
            wait_idx(b, b)
            pltpu.async_copy(g_hbm.at[colv.at[b]], gbuf.at[b], gsem.at[b])

        def round_body(i, _):
            for b in range(NBUF):
                k = i * NBUF + b
                bg = (b + NBUF - 1) % NBUF

                @pl.when(k + NBUF - 1 < NCH)
                def _():
                    wait_idx(bg, k + NBUF - 1)
                    pltpu.async_copy(g_hbm.at[colv.at[bg]], gbuf.at[bg],
                                     gsem.at[bg])
                pltpu.make_async_copy(g_hbm.at[colv.at[b]], gbuf.at[b],
                                      gsem.at[b]).wait()
                fixup(b)
                pltpu.sync_copy(gbuf.at[b], table.at[locv.at[b]], add=True)

                @pl.when(k + NBUF < NCH)
                def _():
                    issue_idx(b, k + NBUF)
            return 0
        lax.fori_loop(0, NCH // NBUF, round_body, 0)
        plsc.subcore_barrier()
        pltpu.sync_copy(table.at[pl.ds(s * HP, HP)],
                        out_hbm.at[pl.ds(c * PH + s * HP, HP)])

    return spmm_kernel(g_tab, row_arr, col_arr)


def _scores_call(fin, users, items, user_num, N, D, B):
    NW = NC * NS
    P = B // NW
    assert B % NW == 0 and P % 16 == 0

    @functools.partial(
        pl.kernel,
        out_type=jax.ShapeDtypeStruct((B,), jnp.float32),
        mesh=_sc_mesh(),
        compiler_params=_SC_PARAMS,
        scratch_types=[
            pltpu.VMEM((P,), jnp.int32),
            pltpu.VMEM((P,), jnp.int32),
            pltpu.VMEM((P, D), jnp.float32),
            pltpu.VMEM((P, D), jnp.float32),
            pltpu.VMEM((P,), jnp.float32),
            pltpu.SemaphoreType.DMA,
        ],
    )
    def score_kernel(fin_hbm, u_hbm, i_hbm, out_hbm, uidx, iidx, ubuf, vbuf,
                     svec, sem):
        c = lax.axis_index("c")
        s = lax.axis_index("s")
        wid = s * NC + c
        base = wid * P
        pltpu.sync_copy(u_hbm.at[pl.ds(base, P)], uidx)
        pltpu.sync_copy(i_hbm.at[pl.ds(base, P)], iidx)
        for j in range(P // 16):
            iidx[pl.ds(j * 16, 16)] = iidx[pl.ds(j * 16, 16)] + user_num
        pltpu.async_copy(fin_hbm.at[uidx], ubuf, sem).wait()
        pltpu.async_copy(fin_hbm.at[iidx], vbuf, sem).wait()
        iota = lax.broadcasted_iota(jnp.int32, (16,), 0)

        def group(j, _):
            acc = jnp.zeros((16,), jnp.float32)
            for k in range(16):
                p = j * 16 + k
                t = ubuf[p, pl.ds(0, 16)] * vbuf[p, pl.ds(0, 16)]
                for q in range(1, D // 16):
                    t = t + ubuf[p, pl.ds(q * 16, 16)] * vbuf[p, pl.ds(q * 16, 16)]
                acc = jnp.where(iota == k, jnp.sum(t), acc)
            svec[pl.ds(j * 16, 16)] = acc
            return 0
        lax.fori_loop(0, P // 16, group, 0)
        pltpu.sync_copy(svec, out_hbm.at[pl.ds(base, P)])

    return score_kernel(fin, users, items)


def _g0_call(deg16, x0, N, D):
    R = 2000
    GRID = N // R

    def body(deg_ref, x_ref, g_ref):
        d = lax.rsqrt(deg_ref[:, 0:1] + 1e-10)
        g_ref[...] = x_ref[...] * d

    return pl.pallas_call(
        body,
        grid=(GRID,),
        in_specs=[
            pl.BlockSpec((R, 16), lambda i: (i, 0)),
            pl.BlockSpec((R, D), lambda i: (i, 0)),
        ],
        out_specs=pl.BlockSpec((R, D), lambda i: (i, 0)),
        out_shape=jax.ShapeDtypeStruct((N, D), jnp.float32),
    )(deg16, x0)


def _layer_call(deg16, acc, epsl, Wmu, bmu, Wlv, blv, Wdec, bdec, N, D,
                need_g):
    R = 2000
    GRID = N // R

    def body(deg_ref, acc_ref, eps_ref, wmu_ref, bmu_ref, wlv_ref, blv_ref,
             wdec_ref, bdec_ref, x_ref, g_ref, part_ref):
        d = lax.rsqrt(deg_ref[:, 0:1] + 1e-10)
        x = acc_ref[...] * d
        x_ref[...] = x
        g_ref[...] = x * d
        mu = jnp.dot(x, wmu_ref[...], preferred_element_type=jnp.float32) \
            + bmu_ref[...]
        lgv = jnp.dot(x, wlv_ref[...], preferred_element_type=jnp.float32) \
            + blv_ref[...]
        elv = jnp.exp(lgv)
        std = jnp.exp(0.5 * lgv)
        z = mu + eps_ref[...] * std
        recon = jnp.dot(z, wdec_ref[...], preferred_element_type=jnp.float32) \
            + bdec_ref[...]
        i = pl.program_id(0)
        part_ref[i, 0] = jnp.sum((recon - x) ** 2)
        part_ref[i, 1] = jnp.sum(1.0 + lgv - mu * mu - elv)

    w_spec = pl.BlockSpec((D, D), lambda i: (0, 0))
    b_spec = pl.BlockSpec((1, D), lambda i: (0, 0))
    x_arr, g_arr, parts = pl.pallas_call(
        body,
        grid=(GRID,),
        in_specs=[
            pl.BlockSpec((R, 16), lambda i: (i, 0)),
            pl.BlockSpec((R, D), lambda i: (i, 0)),
            pl.BlockSpec((R, D), lambda i: (i, 0)),
            w_spec, b_spec, w_spec, b_spec, w_spec, b_spec,
        ],
        out_specs=[
            pl.BlockSpec((R, D), lambda i: (i, 0)),
            pl.BlockSpec((R, D), lambda i: (i, 0)),
            pl.BlockSpec((GRID, 2), lambda i: (0, 0), memory_space=pltpu.SMEM),
        ],
        out_shape=[
            jax.ShapeDtypeStruct((N, D), jnp.float32),
            jax.ShapeDtypeStruct((N, D), jnp.float32),
            jax.ShapeDtypeStruct((GRID, 2), jnp.float32),
        ],
    )(deg16, acc, epsl, Wmu, bmu.reshape(1, D), Wlv, blv.reshape(1, D),
      Wdec, bdec.reshape(1, D))
    del need_g
    return x_arr, g_arr, parts


def _final_call(deg16, acc3, x0, x1, x2, eps2, eps3, W2, W3, attW, attb, N, D):
    R = 2000
    GRID = N // R
    LP1 = 4

    def vae(x, eps_blk, wmu, bmu, wlv, blv, wdec, bdec):
        mu = jnp.dot(x, wmu, preferred_element_type=jnp.float32) + bmu
        lgv = jnp.dot(x, wlv, preferred_element_type=jnp.float32) + blv
        elv = jnp.exp(lgv)
        z = mu + eps_blk * jnp.exp(0.5 * lgv)
        recon = jnp.dot(z, wdec, preferred_element_type=jnp.float32) + bdec
        return jnp.sum((recon - x) ** 2), jnp.sum(1.0 + lgv - mu * mu - elv)

    def body(deg_ref, acc_ref, x0_ref, x1_ref, x2_ref, eps2_ref, eps3_ref,
             wmu2, bmu2, wlv2, blv2, wdec2, bdec2,
             wmu3, bmu3, wlv3, blv3, wdec3, bdec3,
             attw_ref, attb_ref, fin_ref, part_ref):
        d = lax.rsqrt(deg_ref[:, 0:1] + 1e-10)
        x3 = acc_ref[...] * d
        r2, t2 = vae(x3, eps2_ref[...], wmu2[...], bmu2[...], wlv2[...],
                     blv2[...], wdec2[...], bdec2[...])
        r3, t3 = vae(x3, eps3_ref[...], wmu3[...], bmu3[...], wlv3[...],
                     blv3[...], wdec3[...], bdec3[...])
        i = pl.program_id(0)
        part_ref[i, 0] = r2
        part_ref[i, 1] = t2
        part_ref[i, 2] = r3
        part_ref[i, 3] = t3
        e0, e1, e2 = x0_ref[...], x1_ref[...], x2_ref[...]
        flat = jnp.concatenate([e0, e1, e2, x3], axis=1)
        logits = jnp.dot(flat, attw_ref[...],
                         preferred_element_type=jnp.float32) + attb_ref[...]
        m = jnp.max(logits, axis=1, keepdims=True)
        p = jnp.exp(logits - m)
        p = p / jnp.sum(p, axis=1, keepdims=True)
        fin_ref[...] = (e0 * p[:, 0:1] + e1 * p[:, 1:2]
                        + e2 * p[:, 2:3] + x3 * p[:, 3:4])

    w_spec = pl.BlockSpec((D, D), lambda i: (0, 0))
    b_spec = pl.BlockSpec((1, D), lambda i: (0, 0))
    r_spec = pl.BlockSpec((R, D), lambda i: (i, 0))
    (Wmu2, bmu2, Wlv2, blv2, Wdec2, bdec2) = W2
    (Wmu3, bmu3, Wlv3, blv3, Wdec3, bdec3) = W3
    fin, parts = pl.pallas_call(
        body,
        grid=(GRID,),
        in_specs=[
            pl.BlockSpec((R, 16), lambda i: (i, 0)),
            r_spec, r_spec, r_spec, r_spec, r_spec, r_spec,
            w_spec, b_spec, w_spec, b_spec, w_spec, b_spec,
            w_spec, b_spec, w_spec, b_spec, w_spec, b_spec,
            pl.BlockSpec((LP1 * D, LP1), lambda i: (0, 0)),
            pl.BlockSpec((1, LP1), lambda i: (0, 0)),
        ],
        out_specs=[
            r_spec,
            pl.BlockSpec((GRID, 4), lambda i: (0, 0), memory_space=pltpu.SMEM),
        ],
        out_shape=[
            jax.ShapeDtypeStruct((N, D), jnp.float32),
            jax.ShapeDtypeStruct((GRID, 4), jnp.float32),
        ],
    )(deg16, acc3, x0, x1, x2, eps2, eps3,
      Wmu2, bmu2.reshape(1, D), Wlv2, blv2.reshape(1, D), Wdec2,
      bdec2.reshape(1, D),
      Wmu3, bmu3.reshape(1, D), Wlv3, blv3.reshape(1, D), Wdec3,
      bdec3.reshape(1, D),
      attW, attb.reshape(1, LP1))
    return fin, parts


def kernel(users, items, edge_index, user_emb, item_emb, Wmu, bmu, Wlv, blv,
           Wdec, bdec, attW, attb, eps):
    user_num = user_emb.shape[0]
    N = user_num + item_emb.shape[0]
    D = user_emb.shape[1]
    E = edge_index.shape[1]
    B = users.shape[0]
    L = Wmu.shape[0] - 1
    HALF = N // 2
    HP = ((HALF + NS - 1) // NS + 7) // 8 * 8
    PH = HP * NS

    row = edge_index[0]
    col = edge_index[1]
    x0 = jnp.concatenate([user_emb, item_emb], axis=0)

    degpad, _ = _degree_call(row, N, E)
    deg16 = jnp.concatenate([degpad[0:HALF], degpad[PH:PH + HALF]], axis=0)

    g = _g0_call(deg16, x0, N, D)
    xs = [x0]
    parts = []
    for l in range(1, L):
        accpad = _spmm_call(g, row, col, N, E, D)
        acc = jnp.concatenate([accpad[0:HALF], accpad[PH:PH + HALF]], axis=0)
        x_l, g, p = _layer_call(deg16, acc, eps[l - 1], Wmu[l - 1], bmu[l - 1],
                                Wlv[l - 1], blv[l - 1], Wdec[l - 1],
                                bdec[l - 1], N, D, True)
        xs.append(x_l)
        parts.append(p)
    accpad = _spmm_call(g, row, col, N, E, D)
    acc3 = jnp.concatenate([accpad[0:HALF], accpad[PH:PH + HALF]], axis=0)
    W2 = (Wmu[L - 1], bmu[L - 1], Wlv[L - 1], blv[L - 1], Wdec[L - 1],
          bdec[L - 1])
    W3 = (Wmu[L], bmu[L], Wlv[L], blv[L], Wdec[L], bdec[L])
    fin, fparts = _final_call(deg16, acc3, xs[0], xs[1], xs[2], eps[L - 1],
                              eps[L], W2, W3, attW, attb, N, D)

    scores = _scores_call(fin, users, items, user_num, N, D, B)

    ND = float(N * D)
    losses = []
    for p in parts:
        rsum = jnp.sum(p[:, 0])
        tsum = jnp.sum(p[:, 1])
        losses.append(rsum / ND - 0.5 * tsum / ND)
    fr = jnp.sum(fparts, axis=0)
    losses.append(fr[0] / ND - 0.5 * fr[1] / ND)
    losses.append(fr[2] / ND - 0.5 * fr[3] / ND)
    loss = jnp.mean(jnp.stack(losses))
    return scores, loss
